# baseline (speedup 1.0000x reference)
import functools

import jax
import jax.numpy as jnp
from jax import lax
from jax.experimental import pallas as pl
from jax.experimental.pallas import tpu as pltpu
from jax.experimental.pallas import tpu_sc as plsc

_DOWN_RATES = [1, 4, 2]
_KNNS = [16, 12, 8]
_DIM_IN = 128
_DIM = 64
_F32 = jnp.float32


_FPS_R = 16


def _fps_body(xyz_ref, xyzs_ref, idx_ref, npoint, B, N):
    R, C = _FPS_R, N // _FPS_R
    ii = (lax.broadcasted_iota(jnp.int32, (R, C), 0) * C
          + lax.broadcasted_iota(jnp.int32, (R, C), 1))
    coords = [[xyz_ref[3 * b + c] for c in range(3)] for b in range(B)]

    def body(i, state):
        new_state = []
        for b in range(B):
            dist, far = state[2 * b], state[2 * b + 1]
            idx_ref[b, i] = far
            cx = xyzs_ref[3 * b + 0, far]
            cy = xyzs_ref[3 * b + 1, far]
            cz = xyzs_ref[3 * b + 2, far]
            px, py, pz = coords[b]
            d = (px - cx) ** 2 + (py - cy) ** 2 + (pz - cz) ** 2
            dist = jnp.minimum(dist, d)
            lmax = jnp.max(dist, axis=1, keepdims=True)
            lidx = jnp.argmax(dist, axis=1)[:, None]
            m = jnp.max(lmax)
            riota = lax.broadcasted_iota(jnp.int32, (R, 1), 0)
            r_star = jnp.min(jnp.where(lmax == m, riota, R))
            far = r_star * C + jnp.sum(
                jnp.where(riota == r_star, lidx, 0))
            new_state += [dist, far]
        return tuple(new_state)

    init = ()
    for b in range(B):
        init += (jnp.full((R, C), 1e10, _F32), jnp.int32(0))
    lax.fori_loop(0, npoint, body, init)


def _fps_pallas(pq, npoint):
    B, _, N = pq.shape
    xyz = pq.reshape(B * 3, _FPS_R, N // _FPS_R)
    xyzs = pq.reshape(B * 3, N)
    return pl.pallas_call(
        functools.partial(_fps_body, npoint=npoint, B=B, N=N),
        in_specs=[
            pl.BlockSpec(memory_space=pltpu.VMEM),
            pl.BlockSpec(memory_space=pltpu.SMEM),
        ],
        out_shape=jax.ShapeDtypeStruct((B, npoint), jnp.int32),
        out_specs=pl.BlockSpec(memory_space=pltpu.SMEM),
    )(xyz, xyzs)


def _knn_body(qt_ref, st_ref, idx_ref, dv_ref, K, KO, T, M):
    q = qt_ref[0]
    s = st_ref[0]
    mm = jnp.dot(q, s, preferred_element_type=_F32)
    q2 = jnp.sum(q * q, axis=1, keepdims=True)
    s2 = jnp.sum(s * s, axis=0, keepdims=True)
    d = -2.0 * mm + q2 + s2
    lane = lax.broadcasted_iota(jnp.int32, (T, M), 1)
    db = (lax.bitcast_convert_type(jnp.maximum(d, 0.0), jnp.int32)
          & jnp.int32(~0x7FF)) | lane
    INF = jnp.int32(0x7F800000)
    kiota = lax.broadcasted_iota(jnp.int32, (T, KO), 1)
    idxs = jnp.zeros((T, KO), jnp.int32)
    dvs = jnp.zeros((T, KO), _F32)
    for k in range(K):
        mk = jnp.min(db, axis=1)
        idxk = mk & jnp.int32(0x7FF)
        val = lax.bitcast_convert_type(mk & jnp.int32(~0x7FF), _F32)
        idxs = jnp.where(kiota == k, idxk[:, None], idxs)
        dvs = jnp.where(kiota == k, val[:, None], dvs)
        db = jnp.where(lane == idxk[:, None], INF, db)
    idx_ref[0] = idxs
    dv_ref[0] = dvs


def _knn_pallas(K, qt16, st16, KO=None):
    B, Nq, _ = qt16.shape
    M = st16.shape[2]
    KO = KO or K
    T = min(1024, Nq)
    grid = (B, Nq // T)
    idx, dv = pl.pallas_call(
        functools.partial(_knn_body, K=K, KO=KO, T=T, M=M),
        grid=grid,
        in_specs=[
            pl.BlockSpec((1, T, 16), lambda b, t: (b, t, 0)),
            pl.BlockSpec((1, 16, M), lambda b, t: (b, 0, 0)),
        ],
        out_specs=[
            pl.BlockSpec((1, T, KO), lambda b, t: (b, t, 0)),
            pl.BlockSpec((1, T, KO), lambda b, t: (b, t, 0)),
        ],
        out_shape=[
            jax.ShapeDtypeStruct((B, Nq, KO), jnp.int32),
            jax.ShapeDtypeStruct((B, Nq, KO), _F32),
        ],
    )(qt16, st16)
    return idx, dv


def _sc_gather(table, idx):
    R, D = table.shape
    G = idx.shape[0]
    NC, NS = 2, 16
    NW = NC * NS
    assert G % NW == 0, (G, NW)
    per_w = G // NW
    chunk = min(128, per_w)
    nchunk = per_w // chunk
    assert per_w % chunk == 0 and chunk % 8 == 0

    mesh = plsc.VectorSubcoreMesh(core_axis_name="c", subcore_axis_name="s")

    @functools.partial(
        pl.kernel, mesh=mesh,
        out_type=jax.ShapeDtypeStruct((G, D), _F32),
        scratch_types=[
            pltpu.VMEM((chunk,), jnp.int32),
            pltpu.VMEM((chunk, D), _F32),
            pltpu.SemaphoreType.DMA,
        ],
    )
    def k(table_hbm, idx_hbm, out_hbm, idx_v, rows_v, sem):
        wid = lax.axis_index("s") * NC + lax.axis_index("c")
        for c in range(nchunk):
            base = wid * per_w + c * chunk
            pltpu.sync_copy(idx_hbm.at[pl.ds(base, chunk)], idx_v)
            pltpu.async_copy(table_hbm.at[idx_v], rows_v, sem).wait()
            pltpu.sync_copy(rows_v, out_hbm.at[pl.ds(base, chunk)])

    return k(table, idx)


def _kvp_body(fst_ref, ps16_ref, wk_ref, bk_ref, wv_ref, bv_ref, tab_ref, Tm):
    fs_t = fst_ref[...]
    k = (jnp.dot(fs_t, wk_ref[...], preferred_element_type=_F32)
         + bk_ref[...])
    v = (jnp.dot(fs_t, wv_ref[...], preferred_element_type=_F32)
         + bv_ref[...])
    ki = lax.bitcast_convert_type(k, jnp.int32)
    vi = lax.bitcast_convert_type(v, jnp.int32)
    rk = (ki + 0x7FFF + ((ki >> 16) & 1)) >> 16
    rv = (vi + 0x7FFF + ((vi >> 16) & 1)) >> 16
    kvw = lax.bitcast_convert_type((rk & 0xFFFF) | (rv << 16), _F32)
    tab_ref[...] = jnp.concatenate(
        [kvw, ps16_ref[...], jnp.zeros((Tm, 48), _F32)], axis=1)


def _kv_table(fst, ps16f, p):
    BM = fst.shape[0]
    Tm = 512
    return pl.pallas_call(
        functools.partial(_kvp_body, Tm=Tm),
        grid=(BM // Tm,),
        in_specs=[
            pl.BlockSpec((Tm, _DIM_IN), lambda t: (t, 0)),
            pl.BlockSpec((Tm, 16), lambda t: (t, 0)),
            pl.BlockSpec((_DIM_IN, _DIM), lambda t: (0, 0)),
            pl.BlockSpec((1, _DIM), lambda t: (0, 0)),
            pl.BlockSpec((_DIM_IN, _DIM), lambda t: (0, 0)),
            pl.BlockSpec((1, _DIM), lambda t: (0, 0)),
        ],
        out_specs=pl.BlockSpec((Tm, 128), lambda t: (t, 0)),
        out_shape=jax.ShapeDtypeStruct((BM, 128), _F32),
    )(fst, ps16f, p['Wk'].T, p['bk'][None, :], p['Wv'].T, p['bv'][None, :])


def _prep_body(f1g_ref, pref_ref, idx3_ref, dv3_ref,
               w1_ref, b1_ref, w2_ref, b2_ref, ws_ref, bs_ref,
               f1t_ref, T, m):
    f1g = f1g_ref[0]
    pref = pref_ref[0]
    idx3 = idx3_ref[0]
    dv3 = dv3_ref[0]
    d = jnp.maximum(dv3, 1e-10)
    recip = 1.0 / d
    lane8 = lax.broadcasted_iota(jnp.int32, (T, 8), 1)
    recip3 = jnp.where(lane8 < 3, recip, 0.0)
    norm = jnp.sum(recip3, axis=1, keepdims=True)
    w = recip3 / norm
    ci = lax.broadcasted_iota(jnp.int32, (T, m), 1)
    wmat = jnp.zeros((T, m), _F32)
    for j in range(3):
        sel = ci == idx3[:, j:j + 1]
        wmat = wmat + jnp.where(sel, w[:, j:j + 1], 0.0)
    proj = jnp.dot(wmat, pref, preferred_element_type=_F32)
    x = jnp.concatenate([f1g, proj], axis=1)
    h = jax.nn.relu(jnp.dot(x, w1_ref[...], preferred_element_type=_F32)
                    + b1_ref[...])
    out = (jnp.dot(h, w2_ref[...], preferred_element_type=_F32) + b2_ref[...]
           + jnp.dot(x, ws_ref[...], preferred_element_type=_F32)
           + bs_ref[...])
    f1t_ref[0] = out


def _prep_pallas(f1g, pre_f, idx3, dv3, p):
    B, Nq, _ = f1g.shape
    m = pre_f.shape[1]
    T = min(512, Nq)
    grid = (B, Nq // T)
    two = 2 * _DIM_IN
    return pl.pallas_call(
        functools.partial(_prep_body, T=T, m=m),
        grid=grid,
        in_specs=[
            pl.BlockSpec((1, T, _DIM_IN), lambda b, t: (b, t, 0)),
            pl.BlockSpec((1, m, _DIM_IN), lambda b, t: (b, 0, 0)),
            pl.BlockSpec((1, T, 8), lambda b, t: (b, t, 0)),
            pl.BlockSpec((1, T, 8), lambda b, t: (b, t, 0)),
            pl.BlockSpec((two, _DIM_IN), lambda b, t: (0, 0)),
            pl.BlockSpec((1, _DIM_IN), lambda b, t: (0, 0)),
            pl.BlockSpec((_DIM_IN, _DIM_IN), lambda b, t: (0, 0)),
            pl.BlockSpec((1, _DIM_IN), lambda b, t: (0, 0)),
            pl.BlockSpec((two, _DIM_IN), lambda b, t: (0, 0)),
            pl.BlockSpec((1, _DIM_IN), lambda b, t: (0, 0)),
        ],
        out_specs=pl.BlockSpec((1, T, _DIM_IN), lambda b, t: (b, t, 0)),
        out_shape=jax.ShapeDtypeStruct((B, Nq, _DIM_IN), _F32),
    )(f1g, pre_f, idx3, dv3,
      p['W1'].T, p['b1'][None, :], p['W2'].T, p['b2'][None, :],
      p['Ws'].T, p['bs'][None, :])


def _dga_body(f1t_ref, pq16_ref, g_ref,
              wq_ref, bq_ref, pw1_ref, pb1_ref, pw2_ref, pb2_ref,
              aw1_ref, ab1_ref, aw2_ref, ab2_ref, ew_ref, eb_ref,
              out_ref, st1_ref, st2_ref, sty_ref, T, K, n_total):
    ph = pl.program_id(0)
    b = pl.program_id(1)
    t = pl.program_id(2)
    TK = T * K
    first = (b == 0) & (t == 0)

    @pl.when((ph == 0) & first)
    def _init():
        st1_ref[...] = jnp.zeros_like(st1_ref)
        st2_ref[...] = jnp.zeros_like(st2_ref)
        sty_ref[...] = jnp.zeros_like(sty_ref)

    @pl.when((ph == 1) & first)
    def _fin1():
        mu = st1_ref[0:1, :] * (1.0 / n_total)
        ex2 = st1_ref[1:2, :] * (1.0 / n_total)
        inv = lax.rsqrt(jnp.maximum(ex2 - mu * mu, 0.0) + 1e-5)
        st1_ref[0:1, :] = mu
        st1_ref[1:2, :] = inv

    @pl.when((ph == 2) & first)
    def _fin2():
        W = aw1_ref[...]
        b2v = ab1_ref[...]
        mu_y = sty_ref[64:65, :] * (1.0 / n_total)
        mean_a1 = jnp.dot(mu_y, W, preferred_element_type=_F32) + b2v
        Z = jnp.dot(sty_ref[0:64, :] * (1.0 / n_total), W,
                    preferred_element_type=_F32)
        e2 = (jnp.sum(W * Z, axis=0, keepdims=True)
              + 2.0 * b2v * (mean_a1 - b2v) + b2v * b2v)
        inv = lax.rsqrt(jnp.maximum(e2 - mean_a1 * mean_a1, 0.0) + 1e-5)
        st2_ref[0:1, :] = mean_a1
        st2_ref[1:2, :] = inv

    def expand(x):
        C = x.shape[1]
        return jnp.broadcast_to(x[:, None, :], (T, K, C)).reshape(TK, C)

    def segsum(x):
        C = x.shape[1]
        return jnp.sum(x.reshape(T, K, C), axis=1)

    pq16 = pq16_ref[0]
    psg = g_ref[0, :, 64:80]
    kvw = lax.bitcast_convert_type(g_ref[0, :, 0:64], jnp.int32)
    kg = lax.bitcast_convert_type(kvw << 16, _F32)
    vg = lax.bitcast_convert_type(kvw & jnp.int32(0xFFFF0000 - (1 << 32)),
                                  _F32)
    pos_rel = expand(pq16) - psg
    pe = (jnp.dot(pos_rel, pw1_ref[...], preferred_element_type=_F32)
          + pb1_ref[...])

    @pl.when(ph == 0)
    def _acc1():
        st1_ref[0:1, :] += jnp.sum(pe, axis=0, keepdims=True)
        st1_ref[1:2, :] += jnp.sum(pe * pe, axis=0, keepdims=True)

    @pl.when(ph > 0)
    def _main():
        x1 = jax.nn.relu((pe - st1_ref[0:1, :]) * st1_ref[1:2, :])
        pos_emb = (jnp.dot(x1, pw2_ref[...], preferred_element_type=_F32)
                   + pb2_ref[...])
        f1t = f1t_ref[0]
        q = (jnp.dot(f1t, wq_ref[...], preferred_element_type=_F32)
             + bq_ref[...])
        y = expand(q) - kg + pos_emb

        @pl.when(ph == 1)
        def _acc2():
            sty_ref[0:64, :] += lax.dot_general(
                y, y, (((0,), (0,)), ((), ())),
                preferred_element_type=_F32)
            sty_ref[64:65, :] += jnp.sum(y, axis=0, keepdims=True)

        @pl.when(ph == 2)
        def _tail():
            a1 = (jnp.dot(y, aw1_ref[...], preferred_element_type=_F32)
                  + ab1_ref[...])
            x2 = jax.nn.relu((a1 - st2_ref[0:1, :]) * st2_ref[1:2, :])
            a2 = (jnp.dot(x2, aw2_ref[...], preferred_element_type=_F32)
                  + ab2_ref[...])
            ex = jnp.exp(a2)
            att = ex / expand(segsum(ex))
            value = vg + pos_emb
            agg = segsum(att * value)
            o = (jnp.dot(agg, ew_ref[...], preferred_element_type=_F32)
                 + eb_ref[...] + f1t)
            out_ref[0] = o


def _dga_pallas(p, f1t, pq16, g, K):
    B, Nq, _ = f1t.shape
    T = min(512, Nq)
    NT = Nq // T
    TK = T * K
    n_total = float(B * Nq * K)
    grid = (3, B, NT)
    g3 = g.reshape(B * NT, TK, 128)
    return pl.pallas_call(
        functools.partial(_dga_body, T=T, K=K, n_total=n_total),
        grid=grid,
        in_specs=[
            pl.BlockSpec((1, T, _DIM_IN), lambda ph, b, t: (b, t, 0)),
            pl.BlockSpec((1, T, 16), lambda ph, b, t: (b, t, 0)),
            pl.BlockSpec((1, TK, 128), lambda ph, b, t, NT=NT: (b * NT + t, 0, 0)),
            pl.BlockSpec((_DIM_IN, _DIM), lambda ph, b, t: (0, 0)),
            pl.BlockSpec((1, _DIM), lambda ph, b, t: (0, 0)),
            pl.BlockSpec((16, _DIM), lambda ph, b, t: (0, 0)),
            pl.BlockSpec((1, _DIM), lambda ph, b, t: (0, 0)),
            pl.BlockSpec((_DIM, _DIM), lambda ph, b, t: (0, 0)),
            pl.BlockSpec((1, _DIM), lambda ph, b, t: (0, 0)),
            pl.BlockSpec((_DIM, 4 * _DIM), lambda ph, b, t: (0, 0)),
            pl.BlockSpec((1, 4 * _DIM), lambda ph, b, t: (0, 0)),
            pl.BlockSpec((4 * _DIM, _DIM), lambda ph, b, t: (0, 0)),
            pl.BlockSpec((1, _DIM), lambda ph, b, t: (0, 0)),
            pl.BlockSpec((_DIM, _DIM_IN), lambda ph, b, t: (0, 0)),
            pl.BlockSpec((1, _DIM_IN), lambda ph, b, t: (0, 0)),
        ],
        out_specs=pl.BlockSpec((1, T, _DIM_IN), lambda ph, b, t: (b, t, 0)),
        out_shape=jax.ShapeDtypeStruct((B, Nq, _DIM_IN), _F32),
        scratch_shapes=[
            pltpu.VMEM((8, _DIM), _F32),
            pltpu.VMEM((8, 4 * _DIM), _F32),
            pltpu.VMEM((72, _DIM), _F32),
        ],
    )(f1t, pq16, g3,
      p['Wq'].T, p['bq'][None, :],
      jnp.pad(p['pos_W1'], ((0, 0), (0, 13))).T, p['pos_b1'][None, :],
      p['pos_W2'].T, p['pos_b2'][None, :],
      p['attn_W1'].T, p['attn_b1'][None, :],
      p['attn_W2'].T, p['attn_b2'][None, :],
      p['end_W'].T, p['end_b'][None, :])


def kernel(pq, fq, ps, fs, params):
    B, _, N = pq.shape
    M = ps.shape[2]

    pq16 = jnp.pad(pq.transpose(0, 2, 1), ((0, 0), (0, 0), (0, 13)))
    ps16 = jnp.pad(ps.transpose(0, 2, 1), ((0, 0), (0, 0), (0, 13)))
    st16 = ps16.transpose(0, 2, 1)
    fqt = fq.transpose(0, 2, 1).reshape(B * N, _DIM_IN)
    pq16f = pq16.reshape(B * N, 16)
    ps16f = ps16.reshape(B * M, 16)
    boff_n = (jnp.arange(B, dtype=jnp.int32) * N)[:, None]
    fpq_tab = jnp.concatenate(
        [fqt, pq16f, jnp.zeros((B * N, 112), _F32)], axis=1)
    fst = fs.transpose(0, 2, 1).reshape(B * M, _DIM_IN)

    idx512 = _fps_pallas(pq, 512)
    fps_idxs = [None, idx512, idx512[:, :256]]

    f512 = (idx512 + boff_n).reshape(-1)
    f256 = (idx512[:, :256] + boff_n).reshape(-1)
    gf_all = _sc_gather(fpq_tab, jnp.concatenate([f512, f256]))
    fps_g = {512: gf_all[:f512.shape[0]], 256: gf_all[f512.shape[0]:]}

    pre_f = None
    pre_pos16 = None
    for i in range(2, -1, -1):
        fi = fps_idxs[i]
        if fi is None:
            pos1_16 = pq16
            f1g = fqt.reshape(B, N, _DIM_IN)
        else:
            nq = fi.shape[1]
            gf = fps_g[nq]
            pos1_16 = gf[:, 128:144].reshape(B, nq, 16)
            f1g = gf[:, :128].reshape(B, nq, _DIM_IN)
        Nq = pos1_16.shape[1]
        if i != 2:
            qt_pre = pos1_16
            st_pre = pre_pos16.transpose(0, 2, 1)
            idx3, dv3 = _knn_pallas(3, qt_pre, st_pre, KO=8)
            f1t = _prep_pallas(f1g, pre_f, idx3, dv3, params['qmlp'][i])
        else:
            f1t = f1g

        K = _KNNS[i]
        p = params['dga'][i]
        idx_knn, _ = _knn_pallas(K, pos1_16, st16)
        kflat = (idx_knn + (jnp.arange(B, dtype=jnp.int32) * M)[:, None, None]
                 ).reshape(-1)
        kvtab = _kv_table(fst, ps16f, p)
        g = _sc_gather(kvtab, kflat)
        pre_f = _dga_pallas(p, f1t, pos1_16, g, K)
        pre_pos16 = pos1_16

    return pre_f.transpose(0, 2, 1)


# speedup vs baseline: 1.0111x; 1.0111x over previous
import functools

import jax
import jax.numpy as jnp
from jax import lax
from jax.experimental import pallas as pl
from jax.experimental.pallas import tpu as pltpu
from jax.experimental.pallas import tpu_sc as plsc

_DOWN_RATES = [1, 4, 2]
_KNNS = [16, 12, 8]
_DIM_IN = 128
_DIM = 64
_F32 = jnp.float32


_FPS_R = 16


def _fps_body(xyz_ref, xyzs_ref, idx_ref, npoint, B, N):
    R, C = _FPS_R, N // _FPS_R
    ii = (lax.broadcasted_iota(jnp.int32, (R, C), 0) * C
          + lax.broadcasted_iota(jnp.int32, (R, C), 1))
    coords = [[xyz_ref[3 * b + c] for c in range(3)] for b in range(B)]

    def body(i, state):
        new_state = []
        for b in range(B):
            dist, far = state[2 * b], state[2 * b + 1]
            idx_ref[b, i] = far
            cx = xyzs_ref[3 * b + 0, far]
            cy = xyzs_ref[3 * b + 1, far]
            cz = xyzs_ref[3 * b + 2, far]
            px, py, pz = coords[b]
            d = (px - cx) ** 2 + (py - cy) ** 2 + (pz - cz) ** 2
            dist = jnp.minimum(dist, d)
            lmax = jnp.max(dist, axis=1, keepdims=True)
            lidx = jnp.argmax(dist, axis=1)[:, None]
            m = jnp.max(lmax)
            riota = lax.broadcasted_iota(jnp.int32, (R, 1), 0)
            r_star = jnp.min(jnp.where(lmax == m, riota, R))
            far = r_star * C + jnp.sum(
                jnp.where(riota == r_star, lidx, 0))
            new_state += [dist, far]
        return tuple(new_state)

    init = ()
    for b in range(B):
        init += (jnp.full((R, C), 1e10, _F32), jnp.int32(0))
    lax.fori_loop(0, npoint, body, init)


def _fps_pallas(pq, npoint):
    B, _, N = pq.shape
    xyz = pq.reshape(B * 3, _FPS_R, N // _FPS_R)
    xyzs = pq.reshape(B * 3, N)
    return pl.pallas_call(
        functools.partial(_fps_body, npoint=npoint, B=B, N=N),
        in_specs=[
            pl.BlockSpec(memory_space=pltpu.VMEM),
            pl.BlockSpec(memory_space=pltpu.SMEM),
        ],
        out_shape=jax.ShapeDtypeStruct((B, npoint), jnp.int32),
        out_specs=pl.BlockSpec(memory_space=pltpu.SMEM),
    )(xyz, xyzs)


def _knn_body(qt_ref, st_ref, idx_ref, dv_ref, K, KO, T, M):
    q = qt_ref[0]
    s = st_ref[0]
    mm = jnp.dot(q, s, preferred_element_type=_F32)
    q2 = jnp.sum(q * q, axis=1, keepdims=True)
    s2 = jnp.sum(s * s, axis=0, keepdims=True)
    d = -2.0 * mm + q2 + s2
    lane = lax.broadcasted_iota(jnp.int32, (T, M), 1)
    db = (lax.bitcast_convert_type(jnp.maximum(d, 0.0), jnp.int32)
          & jnp.int32(~0x7FF)) | lane
    INF = jnp.int32(0x7F800000)
    kiota = lax.broadcasted_iota(jnp.int32, (T, KO), 1)
    idxs = jnp.zeros((T, KO), jnp.int32)
    dvs = jnp.zeros((T, KO), _F32)
    for k in range(K):
        mk = jnp.min(db, axis=1)
        idxk = mk & jnp.int32(0x7FF)
        val = lax.bitcast_convert_type(mk & jnp.int32(~0x7FF), _F32)
        idxs = jnp.where(kiota == k, idxk[:, None], idxs)
        dvs = jnp.where(kiota == k, val[:, None], dvs)
        if k < K - 1:
            db = jnp.where(lane == idxk[:, None], INF, db)
    idx_ref[0] = idxs
    dv_ref[0] = dvs


def _knn_pallas(K, qt16, st16, KO=None):
    B, Nq, _ = qt16.shape
    M = st16.shape[2]
    KO = KO or K
    T = min(512, Nq)
    grid = (B, Nq // T)
    idx, dv = pl.pallas_call(
        functools.partial(_knn_body, K=K, KO=KO, T=T, M=M),
        grid=grid,
        in_specs=[
            pl.BlockSpec((1, T, 16), lambda b, t: (b, t, 0)),
            pl.BlockSpec((1, 16, M), lambda b, t: (b, 0, 0)),
        ],
        out_specs=[
            pl.BlockSpec((1, T, KO), lambda b, t: (b, t, 0)),
            pl.BlockSpec((1, T, KO), lambda b, t: (b, t, 0)),
        ],
        out_shape=[
            jax.ShapeDtypeStruct((B, Nq, KO), jnp.int32),
            jax.ShapeDtypeStruct((B, Nq, KO), _F32),
        ],
    )(qt16, st16)
    return idx, dv


def _sc_gather(table, idx):
    R, D = table.shape
    G = idx.shape[0]
    NC, NS = 2, 16
    NW = NC * NS
    assert G % NW == 0, (G, NW)
    per_w = G // NW
    chunk = min(128, per_w)
    nchunk = per_w // chunk
    assert per_w % chunk == 0 and chunk % 8 == 0

    mesh = plsc.VectorSubcoreMesh(core_axis_name="c", subcore_axis_name="s")

    @functools.partial(
        pl.kernel, mesh=mesh,
        out_type=jax.ShapeDtypeStruct((G, D), _F32),
        scratch_types=[
            pltpu.VMEM((chunk,), jnp.int32),
            pltpu.VMEM((chunk, D), _F32),
            pltpu.SemaphoreType.DMA,
        ],
    )
    def k(table_hbm, idx_hbm, out_hbm, idx_v, rows_v, sem):
        wid = lax.axis_index("s") * NC + lax.axis_index("c")
        for c in range(nchunk):
            base = wid * per_w + c * chunk
            pltpu.sync_copy(idx_hbm.at[pl.ds(base, chunk)], idx_v)
            pltpu.async_copy(table_hbm.at[idx_v], rows_v, sem).wait()
            pltpu.sync_copy(rows_v, out_hbm.at[pl.ds(base, chunk)])

    return k(table, idx)


def _kvp_body(fst_ref, ps16_ref, wk_ref, bk_ref, wv_ref, bv_ref, tab_ref, Tm):
    fs_t = fst_ref[...]
    k = (jnp.dot(fs_t, wk_ref[...], preferred_element_type=_F32)
         + bk_ref[...])
    v = (jnp.dot(fs_t, wv_ref[...], preferred_element_type=_F32)
         + bv_ref[...])
    ki = lax.bitcast_convert_type(k, jnp.int32)
    vi = lax.bitcast_convert_type(v, jnp.int32)
    rk = (ki + 0x7FFF + ((ki >> 16) & 1)) >> 16
    rv = (vi + 0x7FFF + ((vi >> 16) & 1)) >> 16
    kvw = lax.bitcast_convert_type((rk & 0xFFFF) | (rv << 16), _F32)
    tab_ref[...] = jnp.concatenate(
        [kvw, ps16_ref[...], jnp.zeros((Tm, 48), _F32)], axis=1)


def _kv_table(fst, ps16f, p):
    BM = fst.shape[0]
    Tm = 512
    return pl.pallas_call(
        functools.partial(_kvp_body, Tm=Tm),
        grid=(BM // Tm,),
        in_specs=[
            pl.BlockSpec((Tm, _DIM_IN), lambda t: (t, 0)),
            pl.BlockSpec((Tm, 16), lambda t: (t, 0)),
            pl.BlockSpec((_DIM_IN, _DIM), lambda t: (0, 0)),
            pl.BlockSpec((1, _DIM), lambda t: (0, 0)),
            pl.BlockSpec((_DIM_IN, _DIM), lambda t: (0, 0)),
            pl.BlockSpec((1, _DIM), lambda t: (0, 0)),
        ],
        out_specs=pl.BlockSpec((Tm, 128), lambda t: (t, 0)),
        out_shape=jax.ShapeDtypeStruct((BM, 128), _F32),
    )(fst, ps16f, p['Wk'].T, p['bk'][None, :], p['Wv'].T, p['bv'][None, :])


def _prep_body(f1g_ref, pref_ref, idx3_ref, dv3_ref,
               w1_ref, b1_ref, w2_ref, b2_ref, ws_ref, bs_ref,
               f1t_ref, T, m):
    f1g = f1g_ref[0]
    pref = pref_ref[0]
    idx3 = idx3_ref[0]
    dv3 = dv3_ref[0]
    d = jnp.maximum(dv3, 1e-10)
    recip = 1.0 / d
    lane8 = lax.broadcasted_iota(jnp.int32, (T, 8), 1)
    recip3 = jnp.where(lane8 < 3, recip, 0.0)
    norm = jnp.sum(recip3, axis=1, keepdims=True)
    w = recip3 / norm
    ci = lax.broadcasted_iota(jnp.int32, (T, m), 1)
    wmat = jnp.zeros((T, m), _F32)
    for j in range(3):
        sel = ci == idx3[:, j:j + 1]
        wmat = wmat + jnp.where(sel, w[:, j:j + 1], 0.0)
    proj = jnp.dot(wmat, pref, preferred_element_type=_F32)
    x = jnp.concatenate([f1g, proj], axis=1)
    h = jax.nn.relu(jnp.dot(x, w1_ref[...], preferred_element_type=_F32)
                    + b1_ref[...])
    out = (jnp.dot(h, w2_ref[...], preferred_element_type=_F32) + b2_ref[...]
           + jnp.dot(x, ws_ref[...], preferred_element_type=_F32)
           + bs_ref[...])
    f1t_ref[0] = out


def _prep_pallas(f1g, pre_f, idx3, dv3, p):
    B, Nq, _ = f1g.shape
    m = pre_f.shape[1]
    T = min(512, Nq)
    grid = (B, Nq // T)
    two = 2 * _DIM_IN
    return pl.pallas_call(
        functools.partial(_prep_body, T=T, m=m),
        grid=grid,
        in_specs=[
            pl.BlockSpec((1, T, _DIM_IN), lambda b, t: (b, t, 0)),
            pl.BlockSpec((1, m, _DIM_IN), lambda b, t: (b, 0, 0)),
            pl.BlockSpec((1, T, 8), lambda b, t: (b, t, 0)),
            pl.BlockSpec((1, T, 8), lambda b, t: (b, t, 0)),
            pl.BlockSpec((two, _DIM_IN), lambda b, t: (0, 0)),
            pl.BlockSpec((1, _DIM_IN), lambda b, t: (0, 0)),
            pl.BlockSpec((_DIM_IN, _DIM_IN), lambda b, t: (0, 0)),
            pl.BlockSpec((1, _DIM_IN), lambda b, t: (0, 0)),
            pl.BlockSpec((two, _DIM_IN), lambda b, t: (0, 0)),
            pl.BlockSpec((1, _DIM_IN), lambda b, t: (0, 0)),
        ],
        out_specs=pl.BlockSpec((1, T, _DIM_IN), lambda b, t: (b, t, 0)),
        out_shape=jax.ShapeDtypeStruct((B, Nq, _DIM_IN), _F32),
    )(f1g, pre_f, idx3, dv3,
      p['W1'].T, p['b1'][None, :], p['W2'].T, p['b2'][None, :],
      p['Ws'].T, p['bs'][None, :])


def _dga_body(f1t_ref, pq16_ref, g_ref,
              wq_ref, bq_ref, pw1_ref, pb1_ref, pw2_ref, pb2_ref,
              aw1_ref, ab1_ref, aw2_ref, ab2_ref, ew_ref, eb_ref,
              out_ref, st1_ref, st2_ref, sty_ref, T, K, n_total):
    ph = pl.program_id(0)
    b = pl.program_id(1)
    t = pl.program_id(2)
    TK = T * K
    first = (b == 0) & (t == 0)

    @pl.when((ph == 0) & first)
    def _init():
        st1_ref[...] = jnp.zeros_like(st1_ref)
        st2_ref[...] = jnp.zeros_like(st2_ref)
        sty_ref[...] = jnp.zeros_like(sty_ref)

    @pl.when((ph == 1) & first)
    def _fin1():
        mu = st1_ref[0:1, :] * (1.0 / n_total)
        ex2 = st1_ref[1:2, :] * (1.0 / n_total)
        inv = lax.rsqrt(jnp.maximum(ex2 - mu * mu, 0.0) + 1e-5)
        st1_ref[0:1, :] = mu
        st1_ref[1:2, :] = inv

    @pl.when((ph == 2) & first)
    def _fin2():
        W = aw1_ref[...]
        b2v = ab1_ref[...]
        mu_y = sty_ref[64:65, :] * (1.0 / n_total)
        mean_a1 = jnp.dot(mu_y, W, preferred_element_type=_F32) + b2v
        Z = jnp.dot(sty_ref[0:64, :] * (1.0 / n_total), W,
                    preferred_element_type=_F32)
        e2 = (jnp.sum(W * Z, axis=0, keepdims=True)
              + 2.0 * b2v * (mean_a1 - b2v) + b2v * b2v)
        inv = lax.rsqrt(jnp.maximum(e2 - mean_a1 * mean_a1, 0.0) + 1e-5)
        st2_ref[0:1, :] = mean_a1
        st2_ref[1:2, :] = inv

    def expand(x):
        C = x.shape[1]
        return jnp.broadcast_to(x[:, None, :], (T, K, C)).reshape(TK, C)

    def segsum(x):
        C = x.shape[1]
        return jnp.sum(x.reshape(T, K, C), axis=1)

    pq16 = pq16_ref[0]
    psg = g_ref[0, :, 64:80]
    kvw = lax.bitcast_convert_type(g_ref[0, :, 0:64], jnp.int32)
    kg = lax.bitcast_convert_type(kvw << 16, _F32)
    vg = lax.bitcast_convert_type(kvw & jnp.int32(0xFFFF0000 - (1 << 32)),
                                  _F32)
    pos_rel = expand(pq16) - psg
    pe = (jnp.dot(pos_rel, pw1_ref[...], preferred_element_type=_F32)
          + pb1_ref[...])

    @pl.when(ph == 0)
    def _acc1():
        st1_ref[0:1, :] += jnp.sum(pe, axis=0, keepdims=True)
        st1_ref[1:2, :] += jnp.sum(pe * pe, axis=0, keepdims=True)

    @pl.when(ph > 0)
    def _main():
        x1 = jax.nn.relu((pe - st1_ref[0:1, :]) * st1_ref[1:2, :])
        pos_emb = (jnp.dot(x1, pw2_ref[...], preferred_element_type=_F32)
                   + pb2_ref[...])
        f1t = f1t_ref[0]
        q = (jnp.dot(f1t, wq_ref[...], preferred_element_type=_F32)
             + bq_ref[...])
        y = expand(q) - kg + pos_emb

        @pl.when(ph == 1)
        def _acc2():
            sty_ref[0:64, :] += lax.dot_general(
                y, y, (((0,), (0,)), ((), ())),
                preferred_element_type=_F32)
            sty_ref[64:65, :] += jnp.sum(y, axis=0, keepdims=True)

        @pl.when(ph == 2)
        def _tail():
            a1 = (jnp.dot(y, aw1_ref[...], preferred_element_type=_F32)
                  + ab1_ref[...])
            x2 = jax.nn.relu((a1 - st2_ref[0:1, :]) * st2_ref[1:2, :])
            a2 = (jnp.dot(x2, aw2_ref[...], preferred_element_type=_F32)
                  + ab2_ref[...])
            ex = jnp.exp(a2)
            att = ex / expand(segsum(ex))
            value = vg + pos_emb
            agg = segsum(att * value)
            o = (jnp.dot(agg, ew_ref[...], preferred_element_type=_F32)
                 + eb_ref[...] + f1t)
            out_ref[0] = o


def _dga_pallas(p, f1t, pq16, g, K):
    B, Nq, _ = f1t.shape
    T = min(512, Nq)
    NT = Nq // T
    TK = T * K
    n_total = float(B * Nq * K)
    grid = (3, B, NT)
    g3 = g.reshape(B * NT, TK, 128)
    return pl.pallas_call(
        functools.partial(_dga_body, T=T, K=K, n_total=n_total),
        grid=grid,
        in_specs=[
            pl.BlockSpec((1, T, _DIM_IN), lambda ph, b, t: (b, t, 0)),
            pl.BlockSpec((1, T, 16), lambda ph, b, t: (b, t, 0)),
            pl.BlockSpec((1, TK, 128), lambda ph, b, t, NT=NT: (b * NT + t, 0, 0)),
            pl.BlockSpec((_DIM_IN, _DIM), lambda ph, b, t: (0, 0)),
            pl.BlockSpec((1, _DIM), lambda ph, b, t: (0, 0)),
            pl.BlockSpec((16, _DIM), lambda ph, b, t: (0, 0)),
            pl.BlockSpec((1, _DIM), lambda ph, b, t: (0, 0)),
            pl.BlockSpec((_DIM, _DIM), lambda ph, b, t: (0, 0)),
            pl.BlockSpec((1, _DIM), lambda ph, b, t: (0, 0)),
            pl.BlockSpec((_DIM, 4 * _DIM), lambda ph, b, t: (0, 0)),
            pl.BlockSpec((1, 4 * _DIM), lambda ph, b, t: (0, 0)),
            pl.BlockSpec((4 * _DIM, _DIM), lambda ph, b, t: (0, 0)),
            pl.BlockSpec((1, _DIM), lambda ph, b, t: (0, 0)),
            pl.BlockSpec((_DIM, _DIM_IN), lambda ph, b, t: (0, 0)),
            pl.BlockSpec((1, _DIM_IN), lambda ph, b, t: (0, 0)),
        ],
        out_specs=pl.BlockSpec((1, T, _DIM_IN), lambda ph, b, t: (b, t, 0)),
        out_shape=jax.ShapeDtypeStruct((B, Nq, _DIM_IN), _F32),
        scratch_shapes=[
            pltpu.VMEM((8, _DIM), _F32),
            pltpu.VMEM((8, 4 * _DIM), _F32),
            pltpu.VMEM((72, _DIM), _F32),
        ],
    )(f1t, pq16, g3,
      p['Wq'].T, p['bq'][None, :],
      jnp.pad(p['pos_W1'], ((0, 0), (0, 13))).T, p['pos_b1'][None, :],
      p['pos_W2'].T, p['pos_b2'][None, :],
      p['attn_W1'].T, p['attn_b1'][None, :],
      p['attn_W2'].T, p['attn_b2'][None, :],
      p['end_W'].T, p['end_b'][None, :])


def kernel(pq, fq, ps, fs, params):
    B, _, N = pq.shape
    M = ps.shape[2]

    pq16 = jnp.pad(pq.transpose(0, 2, 1), ((0, 0), (0, 0), (0, 13)))
    ps16 = jnp.pad(ps.transpose(0, 2, 1), ((0, 0), (0, 0), (0, 13)))
    st16 = ps16.transpose(0, 2, 1)
    fqt = fq.transpose(0, 2, 1).reshape(B * N, _DIM_IN)
    pq16f = pq16.reshape(B * N, 16)
    ps16f = ps16.reshape(B * M, 16)
    boff_n = (jnp.arange(B, dtype=jnp.int32) * N)[:, None]
    fpq_tab = jnp.concatenate(
        [fqt, pq16f, jnp.zeros((B * N, 112), _F32)], axis=1)
    fst = fs.transpose(0, 2, 1).reshape(B * M, _DIM_IN)

    idx512 = _fps_pallas(pq, 512)
    fps_idxs = [None, idx512, idx512[:, :256]]

    f512 = (idx512 + boff_n).reshape(-1)
    f256 = (idx512[:, :256] + boff_n).reshape(-1)
    gf_all = _sc_gather(fpq_tab, jnp.concatenate([f512, f256]))
    fps_g = {512: gf_all[:f512.shape[0]], 256: gf_all[f512.shape[0]:]}

    pre_f = None
    pre_pos16 = None
    for i in range(2, -1, -1):
        fi = fps_idxs[i]
        if fi is None:
            pos1_16 = pq16
            f1g = fqt.reshape(B, N, _DIM_IN)
        else:
            nq = fi.shape[1]
            gf = fps_g[nq]
            pos1_16 = gf[:, 128:144].reshape(B, nq, 16)
            f1g = gf[:, :128].reshape(B, nq, _DIM_IN)
        Nq = pos1_16.shape[1]
        if i != 2:
            qt_pre = pos1_16
            st_pre = pre_pos16.transpose(0, 2, 1)
            idx3, dv3 = _knn_pallas(3, qt_pre, st_pre, KO=8)
            f1t = _prep_pallas(f1g, pre_f, idx3, dv3, params['qmlp'][i])
        else:
            f1t = f1g

        K = _KNNS[i]
        p = params['dga'][i]
        idx_knn, _ = _knn_pallas(K, pos1_16, st16)
        kflat = (idx_knn + (jnp.arange(B, dtype=jnp.int32) * M)[:, None, None]
                 ).reshape(-1)
        kvtab = _kv_table(fst, ps16f, p)
        g = _sc_gather(kvtab, kflat)
        pre_f = _dga_pallas(p, f1t, pos1_16, g, K)
        pre_pos16 = pos1_16

    return pre_f.transpose(0, 2, 1)
